# Initial kernel scaffold; baseline (speedup 1.0000x reference)
import jax
import jax.numpy as jnp
from jax.experimental import pallas as pl


def kernel(z, pos_embed):
    raise NotImplementedError("write your pallas kernel here")


# speedup vs baseline: 1.0562x; 1.0562x over previous
import jax
import jax.numpy as jnp
from jax import lax
from jax.experimental import pallas as pl
from jax.experimental.pallas import tpu as pltpu
from jax.experimental.pallas import tpu_sc as plsc

SEQ_LEN = 2048
N_BATCH = 32
D_MODEL = 1024

NUM_WORKERS = 32
ROWS_PER_W = SEQ_LEN // NUM_WORKERS


def _sc_broadcast_body(pe_hbm, out_hbm, rows_v, sem_in, sem_out):
    c = lax.axis_index("c")
    s = lax.axis_index("s")
    wid = s * 2 + c
    base = wid * ROWS_PER_W

    pltpu.async_copy(pe_hbm.at[pl.ds(base, ROWS_PER_W)], rows_v, sem_in).wait()

    out_base = base * N_BATCH

    def fire(i):
        for n in range(N_BATCH):
            pltpu.async_copy(rows_v.at[i], out_hbm.at[out_base + i * N_BATCH + n],
                             sem_out)

    def drain():
        for _ in range(N_BATCH):
            pltpu.make_async_copy(rows_v.at[0], out_hbm.at[out_base], sem_out).wait()

    fire(0)

    @pl.loop(1, ROWS_PER_W)
    def _(i):
        fire(i)
        drain()

    drain()


def kernel(z, pos_embed):
    del z
    mesh = plsc.VectorSubcoreMesh(core_axis_name="c", subcore_axis_name="s")
    out2d = pl.kernel(
        _sc_broadcast_body,
        out_type=jax.ShapeDtypeStruct((SEQ_LEN * N_BATCH, D_MODEL), jnp.float32),
        mesh=mesh,
        scratch_types=[
            pltpu.VMEM((ROWS_PER_W, D_MODEL), jnp.float32),
            pltpu.SemaphoreType.DMA,
            pltpu.SemaphoreType.DMA,
        ],
    )(pos_embed)
    return out2d.reshape(SEQ_LEN, N_BATCH, D_MODEL)


# speedup vs baseline: 1.0701x; 1.0131x over previous
import jax
import jax.numpy as jnp
from jax import lax
from jax.experimental import pallas as pl
from jax.experimental.pallas import tpu as pltpu
from jax.experimental.pallas import tpu_sc as plsc

SEQ_LEN = 2048
N_BATCH = 32
D_MODEL = 1024

NUM_WORKERS = 32
ROWS_PER_W = SEQ_LEN // NUM_WORKERS


def _sc_broadcast_body(pe_hbm, out_hbm, rows_v, sem_in, sem_out):
    c = lax.axis_index("c")
    s = lax.axis_index("s")
    wid = s * 2 + c
    base = wid * ROWS_PER_W

    pltpu.async_copy(pe_hbm.at[pl.ds(base, ROWS_PER_W)], rows_v, sem_in).wait()

    for n in range(N_BATCH):
        pltpu.async_copy(rows_v, out_hbm.at[pl.ds(base, ROWS_PER_W), n], sem_out)
    for _ in range(N_BATCH):
        pltpu.make_async_copy(rows_v, out_hbm.at[pl.ds(base, ROWS_PER_W), 0],
                              sem_out).wait()


def kernel(z, pos_embed):
    del z
    mesh = plsc.VectorSubcoreMesh(core_axis_name="c", subcore_axis_name="s")
    return pl.kernel(
        _sc_broadcast_body,
        out_type=jax.ShapeDtypeStruct((SEQ_LEN, N_BATCH, D_MODEL), jnp.float32),
        mesh=mesh,
        scratch_types=[
            pltpu.VMEM((ROWS_PER_W, D_MODEL), jnp.float32),
            pltpu.SemaphoreType.DMA,
            pltpu.SemaphoreType.DMA,
        ],
    )(pos_embed)
